# baseline (speedup 1.0000x reference)
import functools

import jax
import jax.numpy as jnp
from jax import lax
from jax.experimental import pallas as pl
from jax.experimental.pallas import tpu as pltpu
from jax.experimental.pallas import tpu_sc as plsc

VOCAB = 1000
HIDDEN = 128
BATCH = 1024
SEQ = 50
N = BATCH * SEQ
NW = 32
BPW = N // NW
CHUNK = 80
NCHUNK = BPW // CHUNK

_mesh = plsc.VectorSubcoreMesh(core_axis_name="c", subcore_axis_name="s")


@functools.partial(
    pl.kernel,
    mesh=_mesh,
    out_type=jax.ShapeDtypeStruct((N, HIDDEN), jnp.float32),
    scratch_types=[
        pltpu.VMEM((BPW,), jnp.int32),
        pltpu.VMEM((CHUNK, HIDDEN), jnp.float32),
        pltpu.SemaphoreType.DMA,
    ],
)
def _gather_x(emb_hbm, idx_hbm, out_hbm, idx_v, rows_v, sem):
    wid = lax.axis_index("s") * 2 + lax.axis_index("c")
    base = wid * BPW
    pltpu.sync_copy(idx_hbm.at[pl.ds(base, BPW)], idx_v)

    def body(g, carry):
        pltpu.async_copy(
            emb_hbm.at[idx_v.at[pl.ds(g * CHUNK, CHUNK)]], rows_v, sem
        ).wait()
        pltpu.sync_copy(rows_v, out_hbm.at[pl.ds(base + g * CHUNK, CHUNK)])
        return carry

    lax.fori_loop(0, NCHUNK, body, 0)


NB = 512
GRID = N // NB


def _proj_body(x_ref, w_ref, b_ref, out_ref):
    out_ref[...] = lax.dot_general(
        x_ref[...], w_ref[...],
        dimension_numbers=(((1,), (1,)), ((), ())),
        preferred_element_type=jnp.float32,
    ) + b_ref[...]


def _proj(x, proj_w, proj_b):
    return pl.pallas_call(
        _proj_body,
        grid=(GRID,),
        in_specs=[
            pl.BlockSpec((NB, HIDDEN), lambda i: (i, 0)),
            pl.BlockSpec((VOCAB, HIDDEN), lambda i: (0, 0)),
            pl.BlockSpec((1, VOCAB), lambda i: (0, 0)),
        ],
        out_specs=pl.BlockSpec((NB, VOCAB), lambda i: (i, 0)),
        out_shape=jax.ShapeDtypeStruct((N, VOCAB), jnp.float32),
    )(x, proj_w, proj_b.reshape(1, VOCAB))


def kernel(input_ids, emb_table, proj_w, proj_b):
    ids = input_ids.reshape(-1).astype(jnp.int32)
    x = _gather_x(emb_table, ids)
    out = _proj(x, proj_w, proj_b)
    return out.reshape(BATCH, SEQ, VOCAB)
